# baseline (speedup 1.0000x reference)
import functools

import jax
import jax.numpy as jnp
from jax import lax
from jax.experimental import pallas as pl
from jax.experimental.pallas import tpu as pltpu
from jax.experimental.pallas import tpu_sc as plsc

L = 2
D = 128
N = 10000
NP = 10240
T = 8
E = 320000

NC = 2
NS = 16
CHUNK = 80
EDGES_PER_W = E // (NC * NS)
NCHUNK = EDGES_PER_W // CHUNK
SBC = 25
ROWS_PER_S = NP // NS
ZROWS = 128
DEGW = 16


def _sc_agg_body(with_deg, h_hbm, src_hbm, dst_hbm, *refs):
    if with_deg:
        (agg_out, deg_out, agg_s, deg_s,
         src_v, dst_v, msg0_v, msg1_v, ones_v, dzb_v, gsem0, gsem1) = refs
    else:
        (agg_out, agg_s, src_v, dst_v, msg0_v, msg1_v, gsem0, gsem1) = refs
    c = lax.axis_index("c")
    s = lax.axis_index("s")
    w = c * NS + s

    z16 = jnp.zeros((16,), jnp.float32)

    def zrow(i, carry):
        msg0_v[i // 8, pl.ds((i % 8) * 16, 16)] = z16
        return carry
    lax.fori_loop(0, CHUNK * 8, zrow, None)

    if with_deg:
        one16 = jnp.ones((16,), jnp.float32)

        def drow(i, carry):
            dzb_v[i, :] = z16
            ones_v[i, :] = one16
            return carry
        lax.fori_loop(0, CHUNK, drow, None)

    for b in range(ROWS_PER_S // CHUNK):
        pltpu.sync_copy(msg0_v, agg_s.at[pl.ds(s * ROWS_PER_S + b * CHUNK, CHUNK)])
        if with_deg:
            pltpu.sync_copy(dzb_v, deg_s.at[pl.ds(s * ROWS_PER_S + b * CHUNK, CHUNK)])
    plsc.subcore_barrier()

    def gather(j, buf, sem):
        return pltpu.async_copy(h_hbm.at[src_v.at[j]], buf, sem)

    def wait_gather(j, buf, sem):
        pltpu.make_async_copy(h_hbm.at[src_v.at[j]], buf, sem).wait()

    def scatter(j, buf):
        pltpu.sync_copy(buf, agg_s.at[dst_v.at[j]], add=True)
        if with_deg:
            pltpu.sync_copy(ones_v, deg_s.at[dst_v.at[j]], add=True)

    def super_block(sb, carry):
        pltpu.sync_copy(src_hbm.at[w, pl.ds(sb * SBC, SBC)], src_v)
        pltpu.sync_copy(dst_hbm.at[w, pl.ds(sb * SBC, SBC)], dst_v)
        gather(0, msg0_v, gsem0)

        def chunk2(i, carry2):
            j0 = 2 * i
            wait_gather(j0, msg0_v, gsem0)
            gather(j0 + 1, msg1_v, gsem1)
            scatter(j0, msg0_v)
            wait_gather(j0 + 1, msg1_v, gsem1)
            gather(j0 + 2, msg0_v, gsem0)
            scatter(j0 + 1, msg1_v)
            return carry2
        lax.fori_loop(0, (SBC - 1) // 2, chunk2, None)
        wait_gather(SBC - 1, msg0_v, gsem0)
        scatter(SBC - 1, msg0_v)
        return carry
    lax.fori_loop(0, NCHUNK // SBC, super_block, None)
    plsc.subcore_barrier()

    row0 = s * ROWS_PER_S
    pltpu.sync_copy(agg_s.at[pl.ds(row0, ROWS_PER_S)],
                    agg_out.at[c, pl.ds(row0, ROWS_PER_S)])
    if with_deg:
        pltpu.sync_copy(deg_s.at[pl.ds(row0, ROWS_PER_S)],
                        deg_out.at[c, pl.ds(row0, ROWS_PER_S)])


def _make_sc_agg(with_deg):
    mesh = plsc.VectorSubcoreMesh(core_axis_name="c", subcore_axis_name="s")
    out_type = [jax.ShapeDtypeStruct((NC, NP, D), jnp.float32)]
    scratch = [
        pltpu.VMEM_SHARED((NP, D), jnp.float32),
    ]
    if with_deg:
        out_type.append(jax.ShapeDtypeStruct((NC, NP, DEGW), jnp.float32))
        scratch.append(pltpu.VMEM_SHARED((NP, DEGW), jnp.float32))
    vmem = [
        pltpu.VMEM((SBC, CHUNK), jnp.int32),
        pltpu.VMEM((SBC, CHUNK), jnp.int32),
        pltpu.VMEM((CHUNK, D), jnp.float32),
        pltpu.VMEM((CHUNK, D), jnp.float32),
    ]
    if with_deg:
        vmem.append(pltpu.VMEM((CHUNK, DEGW), jnp.float32))
        vmem.append(pltpu.VMEM((CHUNK, DEGW), jnp.float32))
    vmem.append(pltpu.SemaphoreType.DMA)
    vmem.append(pltpu.SemaphoreType.DMA)
    if with_deg:
        scratch_types = [scratch[0], scratch[1]] + vmem
    else:
        scratch_types = [scratch[0]] + vmem
    return pl.kernel(
        functools.partial(_sc_agg_body, with_deg),
        out_type=tuple(out_type),
        mesh=mesh,
        scratch_types=scratch_types,
        compiler_params=pltpu.CompilerParams(use_tc_tiling_on_sc=False),
    )


_sc_agg_deg = _make_sc_agg(True)
_sc_agg = _make_sc_agg(False)


def _evolve_body(W0_ref, gW_ref, gU_ref, gb_ref, out_ref):
    Ws = [W0_ref[0], W0_ref[1]]
    for _t in range(T):
        for l in range(L):
            Wl = Ws[l]
            Z = jax.nn.sigmoid(gW_ref[l, 0] @ Wl + gU_ref[l, 0] @ Wl + gb_ref[l, 0])
            R = jax.nn.sigmoid(gW_ref[l, 1] @ Wl + gU_ref[l, 1] @ Wl + gb_ref[l, 1])
            Htil = jnp.tanh(gW_ref[l, 2] @ Wl + gU_ref[l, 2] @ (R * Wl) + gb_ref[l, 2])
            Ws[l] = (1.0 - Z) * Wl + Z * Htil
    out_ref[0] = Ws[0]
    out_ref[1] = Ws[1]


def _evolve(gcn_W0, gru_W, gru_U, gru_b):
    return pl.pallas_call(
        _evolve_body,
        out_shape=jax.ShapeDtypeStruct((L, D, D), jnp.float32),
    )(gcn_W0, gru_W, gru_U, gru_b)


BN = 1280


def _mid_body(agg_ref, deg_ref, w_ref, out_ref):
    p = agg_ref[0] + agg_ref[1]
    d = jnp.maximum(deg_ref[0, :, 0:1] + deg_ref[1, :, 0:1], 1.0)
    out_ref[...] = jnp.maximum(jnp.dot(p / d, w_ref[0],
                                       preferred_element_type=jnp.float32), 0.0)


def _mid(agg_p, deg_p, Wf):
    return pl.pallas_call(
        _mid_body,
        grid=(NP // BN,),
        in_specs=[
            pl.BlockSpec((NC, BN, D), lambda i: (0, i, 0)),
            pl.BlockSpec((NC, BN, DEGW), lambda i: (0, i, 0)),
            pl.BlockSpec((1, D, D), lambda i: (0, 0, 0)),
        ],
        out_specs=pl.BlockSpec((BN, D), lambda i: (i, 0)),
        out_shape=jax.ShapeDtypeStruct((NP, D), jnp.float32),
    )(agg_p, deg_p, Wf)


def _final_body(agg_ref, deg_ref, w_ref, g_ref, b_ref, lw_ref, lb_ref, out_ref):
    p = agg_ref[0] + agg_ref[1]
    d = jnp.maximum(deg_ref[0, :, 0:1] + deg_ref[1, :, 0:1], 1.0)
    h = jnp.maximum(jnp.dot(p / d, w_ref[0],
                            preferred_element_type=jnp.float32), 0.0)
    mu = jnp.mean(h, axis=1, keepdims=True)
    var = jnp.mean((h - mu) * (h - mu), axis=1, keepdims=True)
    hn = (h - mu) * lax.rsqrt(var + 1e-5) * g_ref[...] + b_ref[...]
    out_ref[...] = jnp.dot(hn, lw_ref[...],
                           preferred_element_type=jnp.float32) + lb_ref[...]


def _final(agg_p, deg_p, Wf, ln_gamma, ln_beta, lin_W, lin_b):
    return pl.pallas_call(
        _final_body,
        grid=(NP // BN,),
        in_specs=[
            pl.BlockSpec((NC, BN, D), lambda i: (0, i, 0)),
            pl.BlockSpec((NC, BN, DEGW), lambda i: (0, i, 0)),
            pl.BlockSpec((1, D, D), lambda i: (1, 0, 0)),
            pl.BlockSpec((1, D), lambda i: (0, 0)),
            pl.BlockSpec((1, D), lambda i: (0, 0)),
            pl.BlockSpec((D, 1), lambda i: (0, 0)),
            pl.BlockSpec((1, 1), lambda i: (0, 0)),
        ],
        out_specs=pl.BlockSpec((BN, 1), lambda i: (i, 0)),
        out_shape=jax.ShapeDtypeStruct((NP, 1), jnp.float32),
    )(agg_p, deg_p, Wf, ln_gamma.reshape(1, D), ln_beta.reshape(1, D),
      lin_W, lin_b.reshape(1, 1))


def kernel(x, edge_index, gcn_W0, gru_W, gru_U, gru_b, ln_gamma, ln_beta, lin_W, lin_b):
    dst = edge_index[1].reshape(NC * NS, NCHUNK, CHUNK)
    src1 = (edge_index[0] * T + (T - 1)).reshape(NC * NS, NCHUNK, CHUNK)
    src2 = edge_index[0].reshape(NC * NS, NCHUNK, CHUNK)
    xf = x.reshape(N * T, D)
    Wf = _evolve(gcn_W0, gru_W, gru_U, gru_b)
    agg_p, deg_p = _sc_agg_deg(xf, src1, dst)
    h1 = _mid(agg_p, deg_p, Wf)
    (agg2_p,) = _sc_agg(h1, src2, dst)
    out = _final(agg2_p, deg_p, Wf, ln_gamma, ln_beta, lin_W, lin_b)
    return out[:N, 0]


# speedup vs baseline: 1.0014x; 1.0014x over previous
import functools

import jax
import jax.numpy as jnp
from jax import lax
from jax.experimental import pallas as pl
from jax.experimental.pallas import tpu as pltpu
from jax.experimental.pallas import tpu_sc as plsc

L = 2
D = 128
N = 10000
NP = 10240
T = 8
E = 320000

NC = 2
NS = 16
CHUNK = 80
EDGES_PER_W = E // (NC * NS)
NCHUNK = EDGES_PER_W // CHUNK
SBC = 25
ROWS_PER_S = NP // NS
ZROWS = 128
DEGW = 16


def _sc_agg_body(with_deg, h_hbm, src_hbm, dst_hbm, *refs):
    if with_deg:
        (agg_out, deg_out, agg_s, deg_s,
         src_v, dst_v, msg0_v, msg1_v, ones_v, dzb_v, gsem0, gsem1) = refs
    else:
        (agg_out, agg_s, src_v, dst_v, msg0_v, msg1_v, gsem0, gsem1) = refs
    c = lax.axis_index("c")
    s = lax.axis_index("s")
    w = c * NS + s

    z16 = jnp.zeros((16,), jnp.float32)

    def zrow(i, carry):
        msg0_v[i // 8, pl.ds((i % 8) * 16, 16)] = z16
        return carry
    lax.fori_loop(0, CHUNK * 8, zrow, None)

    if with_deg:
        one16 = jnp.ones((16,), jnp.float32)

        def drow(i, carry):
            dzb_v[i, :] = z16
            ones_v[i, :] = one16
            return carry
        lax.fori_loop(0, CHUNK, drow, None)

    for b in range(ROWS_PER_S // CHUNK):
        pltpu.async_copy(msg0_v, agg_s.at[pl.ds(s * ROWS_PER_S + b * CHUNK, CHUNK)],
                         gsem0)
        if with_deg:
            pltpu.async_copy(dzb_v, deg_s.at[pl.ds(s * ROWS_PER_S + b * CHUNK, CHUNK)],
                             gsem1)
    for b in range(ROWS_PER_S // CHUNK):
        pltpu.make_async_copy(msg0_v, agg_s.at[pl.ds(s * ROWS_PER_S + b * CHUNK, CHUNK)],
                              gsem0).wait()
        if with_deg:
            pltpu.make_async_copy(dzb_v, deg_s.at[pl.ds(s * ROWS_PER_S + b * CHUNK, CHUNK)],
                                  gsem1).wait()
    plsc.subcore_barrier()

    def gather(j, buf, sem):
        return pltpu.async_copy(h_hbm.at[src_v.at[j]], buf, sem)

    def wait_gather(j, buf, sem):
        pltpu.make_async_copy(h_hbm.at[src_v.at[j]], buf, sem).wait()

    def scatter(j, buf):
        pltpu.sync_copy(buf, agg_s.at[dst_v.at[j]], add=True)
        if with_deg:
            pltpu.sync_copy(ones_v, deg_s.at[dst_v.at[j]], add=True)

    def super_block(sb, carry):
        pltpu.sync_copy(src_hbm.at[w, pl.ds(sb * SBC, SBC)], src_v)
        pltpu.sync_copy(dst_hbm.at[w, pl.ds(sb * SBC, SBC)], dst_v)
        gather(0, msg0_v, gsem0)

        def chunk2(i, carry2):
            j0 = 2 * i
            wait_gather(j0, msg0_v, gsem0)
            gather(j0 + 1, msg1_v, gsem1)
            scatter(j0, msg0_v)
            wait_gather(j0 + 1, msg1_v, gsem1)
            gather(j0 + 2, msg0_v, gsem0)
            scatter(j0 + 1, msg1_v)
            return carry2
        lax.fori_loop(0, (SBC - 1) // 2, chunk2, None)
        wait_gather(SBC - 1, msg0_v, gsem0)
        scatter(SBC - 1, msg0_v)
        return carry
    lax.fori_loop(0, NCHUNK // SBC, super_block, None)
    plsc.subcore_barrier()

    row0 = s * ROWS_PER_S
    pltpu.sync_copy(agg_s.at[pl.ds(row0, ROWS_PER_S)],
                    agg_out.at[c, pl.ds(row0, ROWS_PER_S)])
    if with_deg:
        pltpu.sync_copy(deg_s.at[pl.ds(row0, ROWS_PER_S)],
                        deg_out.at[c, pl.ds(row0, ROWS_PER_S)])


def _make_sc_agg(with_deg):
    mesh = plsc.VectorSubcoreMesh(core_axis_name="c", subcore_axis_name="s")
    out_type = [jax.ShapeDtypeStruct((NC, NP, D), jnp.float32)]
    scratch = [
        pltpu.VMEM_SHARED((NP, D), jnp.float32),
    ]
    if with_deg:
        out_type.append(jax.ShapeDtypeStruct((NC, NP, DEGW), jnp.float32))
        scratch.append(pltpu.VMEM_SHARED((NP, DEGW), jnp.float32))
    vmem = [
        pltpu.VMEM((SBC, CHUNK), jnp.int32),
        pltpu.VMEM((SBC, CHUNK), jnp.int32),
        pltpu.VMEM((CHUNK, D), jnp.float32),
        pltpu.VMEM((CHUNK, D), jnp.float32),
    ]
    if with_deg:
        vmem.append(pltpu.VMEM((CHUNK, DEGW), jnp.float32))
        vmem.append(pltpu.VMEM((CHUNK, DEGW), jnp.float32))
    vmem.append(pltpu.SemaphoreType.DMA)
    vmem.append(pltpu.SemaphoreType.DMA)
    if with_deg:
        scratch_types = [scratch[0], scratch[1]] + vmem
    else:
        scratch_types = [scratch[0]] + vmem
    return pl.kernel(
        functools.partial(_sc_agg_body, with_deg),
        out_type=tuple(out_type),
        mesh=mesh,
        scratch_types=scratch_types,
        compiler_params=pltpu.CompilerParams(use_tc_tiling_on_sc=False),
    )


_sc_agg_deg = _make_sc_agg(True)
_sc_agg = _make_sc_agg(False)


def _evolve_body(W0_ref, gW_ref, gU_ref, gb_ref, out_ref):
    Ws = [W0_ref[0], W0_ref[1]]
    for _t in range(T):
        for l in range(L):
            Wl = Ws[l]
            Z = jax.nn.sigmoid(gW_ref[l, 0] @ Wl + gU_ref[l, 0] @ Wl + gb_ref[l, 0])
            R = jax.nn.sigmoid(gW_ref[l, 1] @ Wl + gU_ref[l, 1] @ Wl + gb_ref[l, 1])
            Htil = jnp.tanh(gW_ref[l, 2] @ Wl + gU_ref[l, 2] @ (R * Wl) + gb_ref[l, 2])
            Ws[l] = (1.0 - Z) * Wl + Z * Htil
    out_ref[0] = Ws[0]
    out_ref[1] = Ws[1]


def _evolve(gcn_W0, gru_W, gru_U, gru_b):
    return pl.pallas_call(
        _evolve_body,
        out_shape=jax.ShapeDtypeStruct((L, D, D), jnp.float32),
    )(gcn_W0, gru_W, gru_U, gru_b)


BN = 1280


def _mid_body(agg_ref, deg_ref, w_ref, out_ref):
    p = agg_ref[0] + agg_ref[1]
    d = jnp.maximum(deg_ref[0, :, 0:1] + deg_ref[1, :, 0:1], 1.0)
    out_ref[...] = jnp.maximum(jnp.dot(p / d, w_ref[0],
                                       preferred_element_type=jnp.float32), 0.0)


def _mid(agg_p, deg_p, Wf):
    return pl.pallas_call(
        _mid_body,
        grid=(NP // BN,),
        in_specs=[
            pl.BlockSpec((NC, BN, D), lambda i: (0, i, 0)),
            pl.BlockSpec((NC, BN, DEGW), lambda i: (0, i, 0)),
            pl.BlockSpec((1, D, D), lambda i: (0, 0, 0)),
        ],
        out_specs=pl.BlockSpec((BN, D), lambda i: (i, 0)),
        out_shape=jax.ShapeDtypeStruct((NP, D), jnp.float32),
    )(agg_p, deg_p, Wf)


def _final_body(agg_ref, deg_ref, w_ref, g_ref, b_ref, lw_ref, lb_ref, out_ref):
    p = agg_ref[0] + agg_ref[1]
    d = jnp.maximum(deg_ref[0, :, 0:1] + deg_ref[1, :, 0:1], 1.0)
    h = jnp.maximum(jnp.dot(p / d, w_ref[0],
                            preferred_element_type=jnp.float32), 0.0)
    mu = jnp.mean(h, axis=1, keepdims=True)
    var = jnp.mean((h - mu) * (h - mu), axis=1, keepdims=True)
    hn = (h - mu) * lax.rsqrt(var + 1e-5) * g_ref[...] + b_ref[...]
    out_ref[...] = jnp.dot(hn, lw_ref[...],
                           preferred_element_type=jnp.float32) + lb_ref[...]


def _final(agg_p, deg_p, Wf, ln_gamma, ln_beta, lin_W, lin_b):
    return pl.pallas_call(
        _final_body,
        grid=(NP // BN,),
        in_specs=[
            pl.BlockSpec((NC, BN, D), lambda i: (0, i, 0)),
            pl.BlockSpec((NC, BN, DEGW), lambda i: (0, i, 0)),
            pl.BlockSpec((1, D, D), lambda i: (1, 0, 0)),
            pl.BlockSpec((1, D), lambda i: (0, 0)),
            pl.BlockSpec((1, D), lambda i: (0, 0)),
            pl.BlockSpec((D, 1), lambda i: (0, 0)),
            pl.BlockSpec((1, 1), lambda i: (0, 0)),
        ],
        out_specs=pl.BlockSpec((BN, 1), lambda i: (i, 0)),
        out_shape=jax.ShapeDtypeStruct((NP, 1), jnp.float32),
    )(agg_p, deg_p, Wf, ln_gamma.reshape(1, D), ln_beta.reshape(1, D),
      lin_W, lin_b.reshape(1, 1))


def kernel(x, edge_index, gcn_W0, gru_W, gru_U, gru_b, ln_gamma, ln_beta, lin_W, lin_b):
    dst = edge_index[1].reshape(NC * NS, NCHUNK, CHUNK)
    src1 = (edge_index[0] * T + (T - 1)).reshape(NC * NS, NCHUNK, CHUNK)
    src2 = edge_index[0].reshape(NC * NS, NCHUNK, CHUNK)
    xf = x.reshape(N * T, D)
    Wf = _evolve(gcn_W0, gru_W, gru_U, gru_b)
    agg_p, deg_p = _sc_agg_deg(xf, src1, dst)
    h1 = _mid(agg_p, deg_p, Wf)
    (agg2_p,) = _sc_agg(h1, src2, dst)
    out = _final(agg2_p, deg_p, Wf, ln_gamma, ln_beta, lin_W, lin_b)
    return out[:N, 0]


# speedup vs baseline: 1.0140x; 1.0125x over previous
import functools

import jax
import jax.numpy as jnp
from jax import lax
from jax.experimental import pallas as pl
from jax.experimental.pallas import tpu as pltpu
from jax.experimental.pallas import tpu_sc as plsc

L = 2
D = 128
N = 10000
NP = 10240
T = 8
E = 320000

NC = 2
NS = 16
CHUNK = 80
EDGES_PER_W = E // (NC * NS)
NCHUNK = EDGES_PER_W // CHUNK
SBC = 25
ROWS_PER_S = NP // NS
ZROWS = 128
DEGW = 16


def _sc_agg_body(with_deg, h_hbm, src_hbm, dst_hbm, *refs):
    if with_deg:
        (agg_out, deg_out, agg_s, deg_s,
         src_v, dst_v, msg0_v, msg1_v, ones_v, dzb_v, gsem0, gsem1) = refs
    else:
        (agg_out, agg_s, src_v, dst_v, msg0_v, msg1_v, gsem0, gsem1) = refs
    c = lax.axis_index("c")
    s = lax.axis_index("s")
    w = c * NS + s

    z16 = jnp.zeros((16,), jnp.float32)

    def zrow(i, carry):
        msg0_v[i // 8, pl.ds((i % 8) * 16, 16)] = z16
        return carry
    lax.fori_loop(0, CHUNK * 8, zrow, None)

    if with_deg:
        one16 = jnp.ones((16,), jnp.float32)

        def drow(i, carry):
            dzb_v[i, :] = z16
            ones_v[i, :] = one16
            return carry
        lax.fori_loop(0, CHUNK, drow, None)

    for b in range(ROWS_PER_S // CHUNK):
        pltpu.async_copy(msg0_v, agg_s.at[pl.ds(s * ROWS_PER_S + b * CHUNK, CHUNK)],
                         gsem0)
        if with_deg:
            pltpu.async_copy(dzb_v, deg_s.at[pl.ds(s * ROWS_PER_S + b * CHUNK, CHUNK)],
                             gsem1)
    for b in range(ROWS_PER_S // CHUNK):
        pltpu.make_async_copy(msg0_v, agg_s.at[pl.ds(s * ROWS_PER_S + b * CHUNK, CHUNK)],
                              gsem0).wait()
        if with_deg:
            pltpu.make_async_copy(dzb_v, deg_s.at[pl.ds(s * ROWS_PER_S + b * CHUNK, CHUNK)],
                                  gsem1).wait()
    plsc.subcore_barrier()

    def gather(j, buf, sem):
        return pltpu.async_copy(h_hbm.at[src_v.at[j]], buf, sem)

    def wait_gather(j, buf, sem):
        pltpu.make_async_copy(h_hbm.at[src_v.at[j]], buf, sem).wait()

    def scatter(j, buf):
        pltpu.sync_copy(buf, agg_s.at[dst_v.at[j]], add=True)
        if with_deg:
            pltpu.sync_copy(ones_v, deg_s.at[dst_v.at[j]], add=True)

    def super_block(sb, carry):
        pltpu.sync_copy(src_hbm.at[w, pl.ds(sb * SBC, SBC)], src_v)
        pltpu.sync_copy(dst_hbm.at[w, pl.ds(sb * SBC, SBC)], dst_v)
        gather(0, msg0_v, gsem0)

        def chunk2(i, carry2):
            j0 = 2 * i
            wait_gather(j0, msg0_v, gsem0)
            gather(j0 + 1, msg1_v, gsem1)
            scatter(j0, msg0_v)
            wait_gather(j0 + 1, msg1_v, gsem1)
            gather(j0 + 2, msg0_v, gsem0)
            scatter(j0 + 1, msg1_v)
            return carry2
        lax.fori_loop(0, (SBC - 1) // 2, chunk2, None)
        wait_gather(SBC - 1, msg0_v, gsem0)
        scatter(SBC - 1, msg0_v)
        return carry
    lax.fori_loop(0, NCHUNK // SBC, super_block, None)
    plsc.subcore_barrier()

    row0 = s * ROWS_PER_S
    pltpu.sync_copy(agg_s.at[pl.ds(row0, ROWS_PER_S)],
                    agg_out.at[c, pl.ds(row0, ROWS_PER_S)])
    if with_deg:
        pltpu.sync_copy(deg_s.at[pl.ds(row0, ROWS_PER_S)],
                        deg_out.at[c, pl.ds(row0, ROWS_PER_S)])


def _make_sc_agg(with_deg):
    mesh = plsc.VectorSubcoreMesh(core_axis_name="c", subcore_axis_name="s")
    out_type = [jax.ShapeDtypeStruct((NC, NP, D), jnp.float32)]
    scratch = [
        pltpu.VMEM_SHARED((NP, D), jnp.float32),
    ]
    if with_deg:
        out_type.append(jax.ShapeDtypeStruct((NC, NP, DEGW), jnp.float32))
        scratch.append(pltpu.VMEM_SHARED((NP, DEGW), jnp.float32))
    vmem = [
        pltpu.VMEM((SBC, CHUNK), jnp.int32),
        pltpu.VMEM((SBC, CHUNK), jnp.int32),
        pltpu.VMEM((CHUNK, D), jnp.float32),
        pltpu.VMEM((CHUNK, D), jnp.float32),
    ]
    if with_deg:
        vmem.append(pltpu.VMEM((CHUNK, DEGW), jnp.float32))
        vmem.append(pltpu.VMEM((CHUNK, DEGW), jnp.float32))
    vmem.append(pltpu.SemaphoreType.DMA)
    vmem.append(pltpu.SemaphoreType.DMA)
    if with_deg:
        scratch_types = [scratch[0], scratch[1]] + vmem
    else:
        scratch_types = [scratch[0]] + vmem
    return pl.kernel(
        functools.partial(_sc_agg_body, with_deg),
        out_type=tuple(out_type),
        mesh=mesh,
        scratch_types=scratch_types,
        compiler_params=pltpu.CompilerParams(use_tc_tiling_on_sc=False),
    )


_sc_agg_deg = _make_sc_agg(True)
_sc_agg = _make_sc_agg(False)


def _evolve_body(W0_ref, gW_ref, gU_ref, gb_ref, out_ref):
    Ws = [W0_ref[0], W0_ref[1]]
    for _t in range(T):
        for l in range(L):
            Wl = Ws[l]
            Z = jax.nn.sigmoid(gW_ref[l, 0] @ Wl + gU_ref[l, 0] @ Wl + gb_ref[l, 0])
            R = jax.nn.sigmoid(gW_ref[l, 1] @ Wl + gU_ref[l, 1] @ Wl + gb_ref[l, 1])
            Htil = jnp.tanh(gW_ref[l, 2] @ Wl + gU_ref[l, 2] @ (R * Wl) + gb_ref[l, 2])
            Ws[l] = (1.0 - Z) * Wl + Z * Htil
    out_ref[0] = Ws[0]
    out_ref[1] = Ws[1]


def _evolve(gcn_W0, gru_W, gru_U, gru_b):
    return pl.pallas_call(
        _evolve_body,
        out_shape=jax.ShapeDtypeStruct((L, D, D), jnp.float32),
    )(gcn_W0, gru_W, gru_U, gru_b)


BN = 1280


def _mid_body(agg_ref, deg_ref, w_ref, out_ref):
    p = agg_ref[0] + agg_ref[1]
    d = jnp.maximum(deg_ref[0, :, 0:1] + deg_ref[1, :, 0:1], 1.0)
    out_ref[...] = jnp.maximum(jnp.dot(p / d, w_ref[0],
                                       preferred_element_type=jnp.float32), 0.0)


def _mid(agg_p, deg_p, Wf):
    return pl.pallas_call(
        _mid_body,
        grid=(NP // BN,),
        in_specs=[
            pl.BlockSpec((NC, BN, D), lambda i: (0, i, 0)),
            pl.BlockSpec((NC, BN, DEGW), lambda i: (0, i, 0)),
            pl.BlockSpec((1, D, D), lambda i: (0, 0, 0)),
        ],
        out_specs=pl.BlockSpec((BN, D), lambda i: (i, 0)),
        out_shape=jax.ShapeDtypeStruct((NP, D), jnp.float32),
    )(agg_p, deg_p, Wf)


def _final_body(agg_ref, deg_ref, w_ref, g_ref, b_ref, lw_ref, lb_ref, out_ref):
    p = agg_ref[0] + agg_ref[1]
    d = jnp.maximum(deg_ref[0, :, 0:1] + deg_ref[1, :, 0:1], 1.0)
    h = jnp.maximum(jnp.dot(p / d, w_ref[0],
                            preferred_element_type=jnp.float32), 0.0)
    mu = jnp.mean(h, axis=1, keepdims=True)
    var = jnp.mean((h - mu) * (h - mu), axis=1, keepdims=True)
    hn = (h - mu) * lax.rsqrt(var + 1e-5) * g_ref[...] + b_ref[...]
    res = jnp.dot(hn, lw_ref[...],
                  preferred_element_type=jnp.float32) + lb_ref[...]
    out_ref[...] = res[:, 0]


BNF = 1024


def _final(agg_p, deg_p, Wf, ln_gamma, ln_beta, lin_W, lin_b):
    return pl.pallas_call(
        _final_body,
        grid=(NP // BNF,),
        in_specs=[
            pl.BlockSpec((NC, BNF, D), lambda i: (0, i, 0)),
            pl.BlockSpec((NC, BNF, DEGW), lambda i: (0, i, 0)),
            pl.BlockSpec((1, D, D), lambda i: (1, 0, 0)),
            pl.BlockSpec((1, D), lambda i: (0, 0)),
            pl.BlockSpec((1, D), lambda i: (0, 0)),
            pl.BlockSpec((D, 1), lambda i: (0, 0)),
            pl.BlockSpec((1, 1), lambda i: (0, 0)),
        ],
        out_specs=pl.BlockSpec((BNF,), lambda i: (i,)),
        out_shape=jax.ShapeDtypeStruct((NP,), jnp.float32),
    )(agg_p, deg_p, Wf, ln_gamma.reshape(1, D), ln_beta.reshape(1, D),
      lin_W, lin_b.reshape(1, 1))


def kernel(x, edge_index, gcn_W0, gru_W, gru_U, gru_b, ln_gamma, ln_beta, lin_W, lin_b):
    dst = edge_index[1].reshape(NC * NS, NCHUNK, CHUNK)
    src1 = (edge_index[0] * T + (T - 1)).reshape(NC * NS, NCHUNK, CHUNK)
    src2 = edge_index[0].reshape(NC * NS, NCHUNK, CHUNK)
    xf = x.reshape(N * T, D)
    Wf = _evolve(gcn_W0, gru_W, gru_U, gru_b)
    agg_p, deg_p = _sc_agg_deg(xf, src1, dst)
    h1 = _mid(agg_p, deg_p, Wf)
    (agg2_p,) = _sc_agg(h1, src2, dst)
    out = _final(agg2_p, deg_p, Wf, ln_gamma, ln_beta, lin_W, lin_b)
    return out[:N]


# speedup vs baseline: 1.0347x; 1.0204x over previous
import functools

import jax
import jax.numpy as jnp
from jax import lax
from jax.experimental import pallas as pl
from jax.experimental.pallas import tpu as pltpu
from jax.experimental.pallas import tpu_sc as plsc

L = 2
D = 128
N = 10000
NP = 10240
T = 8
E = 320000

NC = 2
NS = 16
CHUNK = 80
EDGES_PER_W = E // (NC * NS)
NCHUNK = EDGES_PER_W // CHUNK
SBC = 25
ROWS_PER_S = NP // NS
ZROWS = 128
DEGW = 16


def _sc_agg_body(with_deg, h_hbm, src_hbm, dst_hbm, *refs):
    if with_deg:
        (agg_out, deg_out, agg_s, deg_s,
         srcA_v, dstA_v, srcB_v, dstB_v, msg0_v, msg1_v, ones_v, dzb_v,
         gsem0, gsem1, isem0, isem1) = refs
    else:
        (agg_out, agg_s, srcA_v, dstA_v, srcB_v, dstB_v, msg0_v, msg1_v,
         gsem0, gsem1, isem0, isem1) = refs
    c = lax.axis_index("c")
    s = lax.axis_index("s")
    w = c * NS + s

    z16 = jnp.zeros((16,), jnp.float32)

    def zrow(i, carry):
        msg0_v[i // 8, pl.ds((i % 8) * 16, 16)] = z16
        return carry
    lax.fori_loop(0, CHUNK * 8, zrow, None)

    if with_deg:
        one16 = jnp.ones((16,), jnp.float32)

        def drow(i, carry):
            dzb_v[i, :] = z16
            ones_v[i, :] = one16
            return carry
        lax.fori_loop(0, CHUNK, drow, None)

    for b in range(ROWS_PER_S // CHUNK):
        pltpu.async_copy(msg0_v, agg_s.at[pl.ds(s * ROWS_PER_S + b * CHUNK, CHUNK)],
                         gsem0)
        if with_deg:
            pltpu.async_copy(dzb_v, deg_s.at[pl.ds(s * ROWS_PER_S + b * CHUNK, CHUNK)],
                             gsem1)
    for b in range(ROWS_PER_S // CHUNK):
        pltpu.make_async_copy(msg0_v, agg_s.at[pl.ds(s * ROWS_PER_S + b * CHUNK, CHUNK)],
                              gsem0).wait()
        if with_deg:
            pltpu.make_async_copy(dzb_v, deg_s.at[pl.ds(s * ROWS_PER_S + b * CHUNK, CHUNK)],
                                  gsem1).wait()
    plsc.subcore_barrier()

    def gather(j, buf, sem, src_v):
        return pltpu.async_copy(h_hbm.at[src_v.at[j]], buf, sem)

    def wait_gather(j, buf, sem, src_v):
        pltpu.make_async_copy(h_hbm.at[src_v.at[j]], buf, sem).wait()

    def scatter(j, buf, dst_v):
        pltpu.sync_copy(buf, agg_s.at[dst_v.at[j]], add=True)
        if with_deg:
            pltpu.sync_copy(ones_v, deg_s.at[dst_v.at[j]], add=True)

    NSB = NCHUNK // SBC
    bufs = [(srcA_v, dstA_v), (srcB_v, dstB_v)]

    pltpu.sync_copy(src_hbm.at[w, pl.ds(0, SBC)], srcA_v)
    pltpu.sync_copy(dst_hbm.at[w, pl.ds(0, SBC)], dstA_v)
    for sb in range(NSB):
        src_v, dst_v = bufs[sb % 2]
        nsrc_v, ndst_v = bufs[(sb + 1) % 2]
        if sb + 1 < NSB:
            pltpu.async_copy(src_hbm.at[w, pl.ds((sb + 1) * SBC, SBC)], nsrc_v, isem0)
            pltpu.async_copy(dst_hbm.at[w, pl.ds((sb + 1) * SBC, SBC)], ndst_v, isem1)
        gather(0, msg0_v, gsem0, src_v)

        def chunk2(i, carry2, src_v=src_v, dst_v=dst_v):
            j0 = 2 * i
            wait_gather(j0, msg0_v, gsem0, src_v)
            gather(j0 + 1, msg1_v, gsem1, src_v)
            scatter(j0, msg0_v, dst_v)
            wait_gather(j0 + 1, msg1_v, gsem1, src_v)
            gather(j0 + 2, msg0_v, gsem0, src_v)
            scatter(j0 + 1, msg1_v, dst_v)
            return carry2
        lax.fori_loop(0, (SBC - 1) // 2, chunk2, None)
        wait_gather(SBC - 1, msg0_v, gsem0, src_v)
        scatter(SBC - 1, msg0_v, dst_v)
        if sb + 1 < NSB:
            pltpu.make_async_copy(src_hbm.at[w, pl.ds((sb + 1) * SBC, SBC)],
                                  nsrc_v, isem0).wait()
            pltpu.make_async_copy(dst_hbm.at[w, pl.ds((sb + 1) * SBC, SBC)],
                                  ndst_v, isem1).wait()
    plsc.subcore_barrier()

    row0 = s * ROWS_PER_S
    pltpu.sync_copy(agg_s.at[pl.ds(row0, ROWS_PER_S)],
                    agg_out.at[c, pl.ds(row0, ROWS_PER_S)])
    if with_deg:
        pltpu.sync_copy(deg_s.at[pl.ds(row0, ROWS_PER_S)],
                        deg_out.at[c, pl.ds(row0, ROWS_PER_S)])


def _make_sc_agg(with_deg):
    mesh = plsc.VectorSubcoreMesh(core_axis_name="c", subcore_axis_name="s")
    out_type = [jax.ShapeDtypeStruct((NC, NP, D), jnp.float32)]
    scratch = [
        pltpu.VMEM_SHARED((NP, D), jnp.float32),
    ]
    if with_deg:
        out_type.append(jax.ShapeDtypeStruct((NC, NP, DEGW), jnp.float32))
        scratch.append(pltpu.VMEM_SHARED((NP, DEGW), jnp.float32))
    vmem = [
        pltpu.VMEM((SBC, CHUNK), jnp.int32),
        pltpu.VMEM((SBC, CHUNK), jnp.int32),
        pltpu.VMEM((SBC, CHUNK), jnp.int32),
        pltpu.VMEM((SBC, CHUNK), jnp.int32),
        pltpu.VMEM((CHUNK, D), jnp.float32),
        pltpu.VMEM((CHUNK, D), jnp.float32),
    ]
    if with_deg:
        vmem.append(pltpu.VMEM((CHUNK, DEGW), jnp.float32))
        vmem.append(pltpu.VMEM((CHUNK, DEGW), jnp.float32))
    vmem.append(pltpu.SemaphoreType.DMA)
    vmem.append(pltpu.SemaphoreType.DMA)
    vmem.append(pltpu.SemaphoreType.DMA)
    vmem.append(pltpu.SemaphoreType.DMA)
    if with_deg:
        scratch_types = [scratch[0], scratch[1]] + vmem
    else:
        scratch_types = [scratch[0]] + vmem
    return pl.kernel(
        functools.partial(_sc_agg_body, with_deg),
        out_type=tuple(out_type),
        mesh=mesh,
        scratch_types=scratch_types,
        compiler_params=pltpu.CompilerParams(use_tc_tiling_on_sc=False),
    )


_sc_agg_deg = _make_sc_agg(True)
_sc_agg = _make_sc_agg(False)


def _evolve_body(W0_ref, gW_ref, gU_ref, gb_ref, out_ref):
    Ws = [W0_ref[0], W0_ref[1]]
    for _t in range(T):
        for l in range(L):
            Wl = Ws[l]
            Z = jax.nn.sigmoid(gW_ref[l, 0] @ Wl + gU_ref[l, 0] @ Wl + gb_ref[l, 0])
            R = jax.nn.sigmoid(gW_ref[l, 1] @ Wl + gU_ref[l, 1] @ Wl + gb_ref[l, 1])
            Htil = jnp.tanh(gW_ref[l, 2] @ Wl + gU_ref[l, 2] @ (R * Wl) + gb_ref[l, 2])
            Ws[l] = (1.0 - Z) * Wl + Z * Htil
    out_ref[0] = Ws[0]
    out_ref[1] = Ws[1]


def _evolve(gcn_W0, gru_W, gru_U, gru_b):
    return pl.pallas_call(
        _evolve_body,
        out_shape=jax.ShapeDtypeStruct((L, D, D), jnp.float32),
    )(gcn_W0, gru_W, gru_U, gru_b)


BN = 1280


def _mid_body(agg_ref, deg_ref, w_ref, out_ref):
    p = agg_ref[0] + agg_ref[1]
    d = jnp.maximum(deg_ref[0, :, 0:1] + deg_ref[1, :, 0:1], 1.0)
    out_ref[...] = jnp.maximum(jnp.dot(p / d, w_ref[0],
                                       preferred_element_type=jnp.float32), 0.0)


def _mid(agg_p, deg_p, Wf):
    return pl.pallas_call(
        _mid_body,
        grid=(NP // BN,),
        in_specs=[
            pl.BlockSpec((NC, BN, D), lambda i: (0, i, 0)),
            pl.BlockSpec((NC, BN, DEGW), lambda i: (0, i, 0)),
            pl.BlockSpec((1, D, D), lambda i: (0, 0, 0)),
        ],
        out_specs=pl.BlockSpec((BN, D), lambda i: (i, 0)),
        out_shape=jax.ShapeDtypeStruct((NP, D), jnp.float32),
    )(agg_p, deg_p, Wf)


def _final_body(agg_ref, deg_ref, w_ref, g_ref, b_ref, lw_ref, lb_ref, out_ref):
    p = agg_ref[0] + agg_ref[1]
    d = jnp.maximum(deg_ref[0, :, 0:1] + deg_ref[1, :, 0:1], 1.0)
    h = jnp.maximum(jnp.dot(p / d, w_ref[0],
                            preferred_element_type=jnp.float32), 0.0)
    mu = jnp.mean(h, axis=1, keepdims=True)
    var = jnp.mean((h - mu) * (h - mu), axis=1, keepdims=True)
    hn = (h - mu) * lax.rsqrt(var + 1e-5) * g_ref[...] + b_ref[...]
    res = jnp.dot(hn, lw_ref[...],
                  preferred_element_type=jnp.float32) + lb_ref[...]
    out_ref[...] = res[:, 0]


BNF = 1024


def _final(agg_p, deg_p, Wf, ln_gamma, ln_beta, lin_W, lin_b):
    return pl.pallas_call(
        _final_body,
        grid=(NP // BNF,),
        in_specs=[
            pl.BlockSpec((NC, BNF, D), lambda i: (0, i, 0)),
            pl.BlockSpec((NC, BNF, DEGW), lambda i: (0, i, 0)),
            pl.BlockSpec((1, D, D), lambda i: (1, 0, 0)),
            pl.BlockSpec((1, D), lambda i: (0, 0)),
            pl.BlockSpec((1, D), lambda i: (0, 0)),
            pl.BlockSpec((D, 1), lambda i: (0, 0)),
            pl.BlockSpec((1, 1), lambda i: (0, 0)),
        ],
        out_specs=pl.BlockSpec((BNF,), lambda i: (i,)),
        out_shape=jax.ShapeDtypeStruct((NP,), jnp.float32),
    )(agg_p, deg_p, Wf, ln_gamma.reshape(1, D), ln_beta.reshape(1, D),
      lin_W, lin_b.reshape(1, 1))


def kernel(x, edge_index, gcn_W0, gru_W, gru_U, gru_b, ln_gamma, ln_beta, lin_W, lin_b):
    dst = edge_index[1].reshape(NC * NS, NCHUNK, CHUNK)
    src1 = (edge_index[0] * T + (T - 1)).reshape(NC * NS, NCHUNK, CHUNK)
    src2 = edge_index[0].reshape(NC * NS, NCHUNK, CHUNK)
    xf = x.reshape(N * T, D)
    Wf = _evolve(gcn_W0, gru_W, gru_U, gru_b)
    agg_p, deg_p = _sc_agg_deg(xf, src1, dst)
    h1 = _mid(agg_p, deg_p, Wf)
    (agg2_p,) = _sc_agg(h1, src2, dst)
    out = _final(agg2_p, deg_p, Wf, ln_gamma, ln_beta, lin_W, lin_b)
    return out[:N]


# speedup vs baseline: 1.0355x; 1.0008x over previous
import functools

import jax
import jax.numpy as jnp
from jax import lax
from jax.experimental import pallas as pl
from jax.experimental.pallas import tpu as pltpu
from jax.experimental.pallas import tpu_sc as plsc

L = 2
D = 128
N = 10000
NP = 10240
T = 8
E = 320000

NC = 2
NS = 16
CHUNK = 80
EDGES_PER_W = E // (NC * NS)
NCHUNK = EDGES_PER_W // CHUNK
SBC = 25
ROWS_PER_S = NP // NS
DEGW = 16


def _sc_agg_body(with_deg, h_hbm, src_hbm, dst_hbm, *refs):
    if with_deg:
        (agg_out, deg_out, agg_s, deg_s,
         srcA_v, dstA_v, srcB_v, dstB_v, msg0_v, msg1_v, ones_v, dzb_v,
         gsem0, gsem1, isem0, isem1) = refs
    else:
        (agg_out, agg_s, srcA_v, dstA_v, srcB_v, dstB_v, msg0_v, msg1_v,
         gsem0, gsem1, isem0, isem1) = refs
    c = lax.axis_index("c")
    s = lax.axis_index("s")
    w = c * NS + s

    z16 = jnp.zeros((16,), jnp.float32)

    def zrow(i, carry):
        msg0_v[i // 8, pl.ds((i % 8) * 16, 16)] = z16
        return carry
    lax.fori_loop(0, CHUNK * 8, zrow, None)

    if with_deg:
        one16 = jnp.ones((16,), jnp.float32)

        def drow(i, carry):
            dzb_v[i, :] = z16
            ones_v[i, :] = one16
            return carry
        lax.fori_loop(0, CHUNK, drow, None)

    for b in range(ROWS_PER_S // CHUNK):
        pltpu.async_copy(msg0_v, agg_s.at[pl.ds(s * ROWS_PER_S + b * CHUNK, CHUNK)],
                         gsem0)
        if with_deg:
            pltpu.async_copy(dzb_v, deg_s.at[pl.ds(s * ROWS_PER_S + b * CHUNK, CHUNK)],
                             gsem1)
    for b in range(ROWS_PER_S // CHUNK):
        pltpu.make_async_copy(msg0_v, agg_s.at[pl.ds(s * ROWS_PER_S + b * CHUNK, CHUNK)],
                              gsem0).wait()
        if with_deg:
            pltpu.make_async_copy(dzb_v, deg_s.at[pl.ds(s * ROWS_PER_S + b * CHUNK, CHUNK)],
                                  gsem1).wait()
    plsc.subcore_barrier()

    def gather(j, buf, sem, src_v):
        return pltpu.async_copy(h_hbm.at[src_v.at[j]], buf, sem)

    def wait_gather(j, buf, sem, src_v):
        pltpu.make_async_copy(h_hbm.at[src_v.at[j]], buf, sem).wait()

    def scatter(j, buf, dst_v):
        pltpu.sync_copy(buf, agg_s.at[dst_v.at[j]], add=True)
        if with_deg:
            pltpu.sync_copy(ones_v, deg_s.at[dst_v.at[j]], add=True)

    NSB = NCHUNK // SBC
    bufs = [(srcA_v, dstA_v), (srcB_v, dstB_v)]

    pltpu.sync_copy(src_hbm.at[w, pl.ds(0, SBC)], srcA_v)
    pltpu.sync_copy(dst_hbm.at[w, pl.ds(0, SBC)], dstA_v)
    for sb in range(NSB):
        src_v, dst_v = bufs[sb % 2]
        nsrc_v, ndst_v = bufs[(sb + 1) % 2]
        if sb + 1 < NSB:
            pltpu.async_copy(src_hbm.at[w, pl.ds((sb + 1) * SBC, SBC)], nsrc_v, isem0)
            pltpu.async_copy(dst_hbm.at[w, pl.ds((sb + 1) * SBC, SBC)], ndst_v, isem1)
        gather(0, msg0_v, gsem0, src_v)

        def chunk2(i, carry2, src_v=src_v, dst_v=dst_v):
            j0 = 2 * i
            wait_gather(j0, msg0_v, gsem0, src_v)
            gather(j0 + 1, msg1_v, gsem1, src_v)
            scatter(j0, msg0_v, dst_v)
            wait_gather(j0 + 1, msg1_v, gsem1, src_v)
            gather(j0 + 2, msg0_v, gsem0, src_v)
            scatter(j0 + 1, msg1_v, dst_v)
            return carry2
        lax.fori_loop(0, (SBC - 1) // 2, chunk2, None)
        wait_gather(SBC - 1, msg0_v, gsem0, src_v)
        scatter(SBC - 1, msg0_v, dst_v)
        if sb + 1 < NSB:
            pltpu.make_async_copy(src_hbm.at[w, pl.ds((sb + 1) * SBC, SBC)],
                                  nsrc_v, isem0).wait()
            pltpu.make_async_copy(dst_hbm.at[w, pl.ds((sb + 1) * SBC, SBC)],
                                  ndst_v, isem1).wait()
    plsc.subcore_barrier()

    row0 = s * ROWS_PER_S
    pltpu.sync_copy(agg_s.at[pl.ds(row0, ROWS_PER_S)],
                    agg_out.at[c, pl.ds(row0, ROWS_PER_S)])
    if with_deg:
        pltpu.sync_copy(deg_s.at[pl.ds(row0, ROWS_PER_S)],
                        deg_out.at[c, pl.ds(row0, ROWS_PER_S)])


def _make_sc_agg(with_deg):
    mesh = plsc.VectorSubcoreMesh(core_axis_name="c", subcore_axis_name="s")
    out_type = [jax.ShapeDtypeStruct((NC, NP, D), jnp.float32)]
    scratch = [
        pltpu.VMEM_SHARED((NP, D), jnp.float32),
    ]
    if with_deg:
        out_type.append(jax.ShapeDtypeStruct((NC, NP, DEGW), jnp.float32))
        scratch.append(pltpu.VMEM_SHARED((NP, DEGW), jnp.float32))
    vmem = [
        pltpu.VMEM((SBC, CHUNK), jnp.int32),
        pltpu.VMEM((SBC, CHUNK), jnp.int32),
        pltpu.VMEM((SBC, CHUNK), jnp.int32),
        pltpu.VMEM((SBC, CHUNK), jnp.int32),
        pltpu.VMEM((CHUNK, D), jnp.float32),
        pltpu.VMEM((CHUNK, D), jnp.float32),
    ]
    if with_deg:
        vmem.append(pltpu.VMEM((CHUNK, DEGW), jnp.float32))
        vmem.append(pltpu.VMEM((CHUNK, DEGW), jnp.float32))
    vmem.append(pltpu.SemaphoreType.DMA)
    vmem.append(pltpu.SemaphoreType.DMA)
    vmem.append(pltpu.SemaphoreType.DMA)
    vmem.append(pltpu.SemaphoreType.DMA)
    if with_deg:
        scratch_types = [scratch[0], scratch[1]] + vmem
    else:
        scratch_types = [scratch[0]] + vmem
    return pl.kernel(
        functools.partial(_sc_agg_body, with_deg),
        out_type=tuple(out_type),
        mesh=mesh,
        scratch_types=scratch_types,
        compiler_params=pltpu.CompilerParams(use_tc_tiling_on_sc=False),
    )


_sc_agg_deg = _make_sc_agg(True)
_sc_agg = _make_sc_agg(False)


def _evolve_body(W0_ref, gW_ref, gU_ref, gb_ref, out_ref):
    Ws = [W0_ref[0], W0_ref[1]]
    for _t in range(T):
        for l in range(L):
            Wl = Ws[l]
            Z = jax.nn.sigmoid(gW_ref[l, 0] @ Wl + gU_ref[l, 0] @ Wl + gb_ref[l, 0])
            R = jax.nn.sigmoid(gW_ref[l, 1] @ Wl + gU_ref[l, 1] @ Wl + gb_ref[l, 1])
            Htil = jnp.tanh(gW_ref[l, 2] @ Wl + gU_ref[l, 2] @ (R * Wl) + gb_ref[l, 2])
            Ws[l] = (1.0 - Z) * Wl + Z * Htil
    out_ref[0] = Ws[0]
    out_ref[1] = Ws[1]


def _evolve(gcn_W0, gru_W, gru_U, gru_b):
    return pl.pallas_call(
        _evolve_body,
        out_shape=jax.ShapeDtypeStruct((L, D, D), jnp.float32),
    )(gcn_W0, gru_W, gru_U, gru_b)


BN = 1280


def _mid_body(agg_ref, deg_ref, w_ref, out_ref):
    p = agg_ref[0] + agg_ref[1]
    d = jnp.maximum(deg_ref[0, :, 0:1] + deg_ref[1, :, 0:1], 1.0)
    out_ref[...] = jnp.maximum(jnp.dot(p / d, w_ref[0],
                                       preferred_element_type=jnp.float32), 0.0)


def _mid(agg_p, deg_p, Wf):
    return pl.pallas_call(
        _mid_body,
        grid=(NP // BN,),
        in_specs=[
            pl.BlockSpec((NC, BN, D), lambda i: (0, i, 0)),
            pl.BlockSpec((NC, BN, DEGW), lambda i: (0, i, 0)),
            pl.BlockSpec((1, D, D), lambda i: (0, 0, 0)),
        ],
        out_specs=pl.BlockSpec((BN, D), lambda i: (i, 0)),
        out_shape=jax.ShapeDtypeStruct((NP, D), jnp.float32),
    )(agg_p, deg_p, Wf)


def _final_body(agg_ref, deg_ref, w_ref, g_ref, b_ref, lw_ref, lb_ref, out_ref):
    p = agg_ref[0] + agg_ref[1]
    d = jnp.maximum(deg_ref[0, :, 0:1] + deg_ref[1, :, 0:1], 1.0)
    h = jnp.maximum(jnp.dot(p / d, w_ref[0],
                            preferred_element_type=jnp.float32), 0.0)
    mu = jnp.mean(h, axis=1, keepdims=True)
    var = jnp.mean((h - mu) * (h - mu), axis=1, keepdims=True)
    hn = (h - mu) * lax.rsqrt(var + 1e-5) * g_ref[...] + b_ref[...]
    res = jnp.dot(hn, lw_ref[...],
                  preferred_element_type=jnp.float32) + lb_ref[...]
    out_ref[...] = res[:, 0]


BNF = 1024


def _final(agg_p, deg_p, Wf, ln_gamma, ln_beta, lin_W, lin_b):
    return pl.pallas_call(
        _final_body,
        grid=(NP // BNF,),
        in_specs=[
            pl.BlockSpec((NC, BNF, D), lambda i: (0, i, 0)),
            pl.BlockSpec((NC, BNF, DEGW), lambda i: (0, i, 0)),
            pl.BlockSpec((1, D, D), lambda i: (1, 0, 0)),
            pl.BlockSpec((1, D), lambda i: (0, 0)),
            pl.BlockSpec((1, D), lambda i: (0, 0)),
            pl.BlockSpec((D, 1), lambda i: (0, 0)),
            pl.BlockSpec((1, 1), lambda i: (0, 0)),
        ],
        out_specs=pl.BlockSpec((BNF,), lambda i: (i,)),
        out_shape=jax.ShapeDtypeStruct((NP,), jnp.float32),
    )(agg_p, deg_p, Wf, ln_gamma.reshape(1, D), ln_beta.reshape(1, D),
      lin_W, lin_b.reshape(1, 1))


def kernel(x, edge_index, gcn_W0, gru_W, gru_U, gru_b, ln_gamma, ln_beta, lin_W, lin_b):
    dst = edge_index[1].reshape(NC * NS, NCHUNK, CHUNK)
    src1 = (edge_index[0] * T + (T - 1)).reshape(NC * NS, NCHUNK, CHUNK)
    src2 = edge_index[0].reshape(NC * NS, NCHUNK, CHUNK)
    xf = x.reshape(N * T, D)
    Wf = _evolve(gcn_W0, gru_W, gru_U, gru_b)
    agg_p, deg_p = _sc_agg_deg(xf, src1, dst)
    h1 = _mid(agg_p, deg_p, Wf)
    (agg2_p,) = _sc_agg(h1, src2, dst)
    out = _final(agg2_p, deg_p, Wf, ln_gamma, ln_beta, lin_W, lin_b)
    return out[:N]
